# baseline (speedup 1.0000x reference)
import functools

import jax
import jax.numpy as jnp
import numpy as np
from jax import lax
from jax.experimental import pallas as pl
from jax.experimental.pallas import tpu as pltpu
from jax.experimental.pallas import tpu_sc as plsc
from jax.sharding import Mesh, PartitionSpec as P

try:
    from jax import shard_map as _shard_map
except ImportError:
    from jax.experimental.shard_map import shard_map as _shard_map

VOCAB = 1000
D = 128
BATCH = 4096
SEQ = 200
N = BATCH * SEQ

NC = 2
NS = 16
NW = NC * NS
CHUNK = 128
NBUF = 4
STAGERS = 5
VPS = VOCAB // STAGERS


def _make_lookup(chunks):
    rpw = chunks * CHUNK
    groups = chunks // NBUF

    @functools.partial(
        pl.kernel,
        out_type=jax.ShapeDtypeStruct((NW * rpw, D), jnp.float32),
        mesh=plsc.VectorSubcoreMesh(core_axis_name="c", subcore_axis_name="s"),
        scratch_types=[
            pltpu.VMEM((chunks, CHUNK), jnp.int32),
            pltpu.VMEM((NBUF, CHUNK, D), jnp.float32),
            pltpu.VMEM((VPS, D), jnp.float32),
            pltpu.VMEM_SHARED((VOCAB, D), jnp.float32),
            pltpu.SemaphoreType.DMA((NBUF,)),
            pltpu.SemaphoreType.DMA((NBUF,)),
        ],
    )
    def _emb_lookup(table_hbm, idx_hbm, out_hbm, idx_v, rows_v, stage_v,
                    table_spm, gsem, ssem):
        sid = lax.axis_index("s")
        wid = sid * NC + lax.axis_index("c")

        @pl.when(sid < STAGERS)
        def _stage():
            pltpu.sync_copy(table_hbm.at[pl.ds(sid * VPS, VPS)], stage_v)
            pltpu.sync_copy(stage_v, table_spm.at[pl.ds(sid * VPS, VPS)])

        pltpu.sync_copy(idx_hbm.at[wid], idx_v)
        plsc.subcore_barrier()
        out_base = wid * rpw

        def gather(j, b):
            pltpu.async_copy(table_spm.at[idx_v.at[j]], rows_v.at[b],
                             gsem.at[b])

        def store(j, b):
            pltpu.async_copy(
                rows_v.at[b], out_hbm.at[pl.ds(out_base + j * CHUNK, CHUNK)],
                ssem.at[b])

        def wait(sem, b):
            pltpu.make_async_copy(
                table_hbm.at[pl.ds(0, CHUNK)], rows_v.at[b], sem.at[b]).wait()

        for b in range(NBUF):
            gather(b, b)

        def body(i, carry):
            for b in range(NBUF):
                j = i * NBUF + b
                wait(gsem, b)
                store(j, b)
                wait(ssem, b)
                gather(j + NBUF, b)
            return carry

        lax.fori_loop(0, groups - 1, body, 0)

        for b in range(NBUF):
            j = (groups - 1) * NBUF + b
            wait(gsem, b)
            store(j, b)
        for b in range(NBUF):
            wait(ssem, b)

    return _emb_lookup


_LOOKUP = {}
for _nd in (1, 2, 4):
    _LOOKUP[_nd] = _make_lookup(N // (_nd * NW * CHUNK))


def kernel(input_seq, embedding_weight):
    nd = 1
    for cand in (4, 2):
        if len(jax.devices()) >= cand and N % (cand * NW * CHUNK) == 0:
            nd = cand
            break
    chunks = N // (nd * NW * CHUNK)
    lookup = _LOOKUP[nd]
    idx = input_seq.reshape(nd, NW, chunks, CHUNK).astype(jnp.int32)
    if nd == 1:
        out = lookup(embedding_weight, idx[0])
    else:
        mesh = Mesh(np.array(jax.devices()[:nd]), ("d",))
        f = _shard_map(
            lambda tbl, ix: lookup(tbl, ix[0]),
            mesh=mesh,
            in_specs=(P(None, None), P("d", None, None, None)),
            out_specs=P("d", None),
            check_vma=False,
        )
        out = f(embedding_weight, idx)
    return out.reshape(BATCH, SEQ, D)


# speedup vs baseline: 2.2380x; 2.2380x over previous
import functools

import jax
import jax.numpy as jnp
from jax import lax
from jax.experimental import pallas as pl
from jax.experimental.pallas import tpu as pltpu
from jax.experimental.pallas import tpu_sc as plsc

VOCAB = 1000
D = 128
BATCH = 4096
SEQ = 200
N = BATCH * SEQ

NC = 2
NS = 16
NW = NC * NS
CHUNK = 128
NBUF = 4
STAGERS = 5
VPS = VOCAB // STAGERS


def _make_lookup(chunks):
    rpw = chunks * CHUNK
    groups = chunks // NBUF

    @functools.partial(
        pl.kernel,
        out_type=jax.ShapeDtypeStruct((NW * rpw, D), jnp.float32),
        mesh=plsc.VectorSubcoreMesh(core_axis_name="c", subcore_axis_name="s"),
        scratch_types=[
            pltpu.VMEM((chunks, CHUNK), jnp.int32),
            pltpu.VMEM((NBUF, CHUNK, D), jnp.float32),
            pltpu.VMEM((VPS, D), jnp.float32),
            pltpu.VMEM_SHARED((VOCAB, D), jnp.float32),
            pltpu.SemaphoreType.DMA((NBUF,)),
            pltpu.SemaphoreType.DMA((NBUF,)),
        ],
    )
    def _emb_lookup(table_hbm, idx_hbm, out_hbm, idx_v, rows_v, stage_v,
                    table_spm, gsem, ssem):
        sid = lax.axis_index("s")
        wid = sid * NC + lax.axis_index("c")

        @pl.when(sid < STAGERS)
        def _stage():
            pltpu.sync_copy(table_hbm.at[pl.ds(sid * VPS, VPS)], stage_v)
            pltpu.sync_copy(stage_v, table_spm.at[pl.ds(sid * VPS, VPS)])

        pltpu.sync_copy(idx_hbm.at[wid], idx_v)
        plsc.subcore_barrier()
        out_base = wid * rpw

        def gather(j, b):
            pltpu.async_copy(table_spm.at[idx_v.at[j]], rows_v.at[b],
                             gsem.at[b])

        def store(j, b):
            pltpu.async_copy(
                rows_v.at[b], out_hbm.at[pl.ds(out_base + j * CHUNK, CHUNK)],
                ssem.at[b])

        def wait(sem, b):
            pltpu.make_async_copy(
                table_hbm.at[pl.ds(0, CHUNK)], rows_v.at[b], sem.at[b]).wait()

        for b in range(NBUF):
            gather(b, b)

        def body(i, carry):
            for b in range(NBUF):
                j = i * NBUF + b
                wait(gsem, b)
                store(j, b)
                wait(ssem, b)
                gather(j + NBUF, b)
            return carry

        lax.fori_loop(0, groups - 1, body, 0)

        for b in range(NBUF):
            j = (groups - 1) * NBUF + b
            wait(gsem, b)
            store(j, b)
        for b in range(NBUF):
            wait(ssem, b)

    return _emb_lookup


_CHUNKS = N // (NW * CHUNK)
_LOOKUP = _make_lookup(_CHUNKS)


def kernel(input_seq, embedding_weight):
    idx = input_seq.reshape(NW, _CHUNKS, CHUNK).astype(jnp.int32)
    out = _LOOKUP(embedding_weight, idx)
    return out.reshape(BATCH, SEQ, D)
